# Initial kernel scaffold; baseline (speedup 1.0000x reference)
import jax
import jax.numpy as jnp
from jax.experimental import pallas as pl


def kernel(x, edge_index, batch, W1, b1, bn_gamma, bn_beta, bn_mean, bn_var, W2, b2, lin_W, lin_b, fin_W, fin_b):
    raise NotImplementedError("write your pallas kernel here")


# speedup vs baseline: 5.6276x; 5.6276x over previous
import functools

import jax
import jax.numpy as jnp
from jax import lax
from jax.experimental import pallas as pl
from jax.experimental.pallas import tpu as pltpu
from jax.experimental.pallas import tpu_sc as plsc

NC = 2
NS = 16
CHUNK = 80


def _sc_aggregate(x, src, dst, zeros, n, d, e):
  ew = e // (NC * NS)
  iters = ew // CHUNK
  npad = ((n + NS * 8 - 1) // (NS * 8)) * NS * 8
  rpt = npad // NS
  rlast = n - (NS - 1) * rpt

  mesh = plsc.VectorSubcoreMesh(core_axis_name="c", subcore_axis_name="s")

  @functools.partial(
      pl.kernel,
      out_type=(
          jax.ShapeDtypeStruct((n, d), jnp.float32),
          jax.ShapeDtypeStruct((n, d), jnp.float32),
      ),
      mesh=mesh,
      scratch_types=[
          pltpu.VMEM((CHUNK,), jnp.int32),
          pltpu.VMEM((CHUNK,), jnp.int32),
          pltpu.VMEM((CHUNK, d), jnp.float32),
          pltpu.VMEM_SHARED((npad, d), jnp.float32),
          pltpu.SemaphoreType.DMA,
      ],
  )
  def agg_kernel(x_hbm, src_hbm, dst_hbm, z_hbm, out0, out1,
                 src_v, dst_v, rows_v, agg_sh, sem):
    c = lax.axis_index("c")
    s = lax.axis_index("s")
    w = c * NS + s

    row0 = s * rpt

    @pl.when(s < NS - 1)
    def _():
      pltpu.sync_copy(z_hbm.at[pl.ds(row0, rpt)],
                      agg_sh.at[pl.ds(row0, rpt)])

    @pl.when(s == NS - 1)
    def _():
      pltpu.sync_copy(z_hbm.at[pl.ds(row0, rlast)],
                      agg_sh.at[pl.ds(row0, rlast)])

    plsc.subcore_barrier()

    def body(i, carry):
      base = w * ew + i * CHUNK
      pltpu.sync_copy(src_hbm.at[pl.ds(base, CHUNK)], src_v)
      pltpu.sync_copy(dst_hbm.at[pl.ds(base, CHUNK)], dst_v)
      pltpu.async_copy(x_hbm.at[src_v], rows_v, sem).wait()
      pltpu.sync_copy(rows_v, agg_sh.at[dst_v], add=True)
      return carry

    lax.fori_loop(0, iters, body, 0)
    plsc.subcore_barrier()

    @pl.when((c == 0) & (s < NS - 1))
    def _():
      pltpu.sync_copy(agg_sh.at[pl.ds(row0, rpt)],
                      out0.at[pl.ds(row0, rpt)])

    @pl.when((c == 0) & (s == NS - 1))
    def _():
      pltpu.sync_copy(agg_sh.at[pl.ds(row0, rlast)],
                      out0.at[pl.ds(row0, rlast)])

    @pl.when((c == 1) & (s < NS - 1))
    def _():
      pltpu.sync_copy(agg_sh.at[pl.ds(row0, rpt)],
                      out1.at[pl.ds(row0, rpt)])

    @pl.when((c == 1) & (s == NS - 1))
    def _():
      pltpu.sync_copy(agg_sh.at[pl.ds(row0, rlast)],
                      out1.at[pl.ds(row0, rlast)])

  return agg_kernel(x, src, dst, zeros)


def _tc_mlp(x, agg0, agg1, batch3, W1f, b1f, W2, b2, lin_W, lin_b,
            fin_Wp, fin_bp, n, d, g, rblk):
  nb = n // rblk

  def body(x_ref, a0_ref, a1_ref, b_ref, w1_ref, b1_ref, w2_ref, b2_ref,
           lw_ref, lb_ref, fw_ref, fb_ref, out_ref, acc):
    i = pl.program_id(0)

    @pl.when(i == 0)
    def _():
      acc[...] = jnp.zeros_like(acc)

    h = x_ref[...] + a0_ref[...] + a1_ref[...]
    h = jnp.maximum(
        jnp.dot(h, w1_ref[...], preferred_element_type=jnp.float32)
        + b1_ref[...], 0.0)
    h = jnp.maximum(
        jnp.dot(h, w2_ref[...], preferred_element_type=jnp.float32)
        + b2_ref[...], 0.0)

    seg = b_ref[0, 0, :]
    mask = (seg[None, :] ==
            lax.broadcasted_iota(jnp.int32, (g, rblk), 0)).astype(jnp.float32)
    acc[...] += jnp.dot(mask, h, preferred_element_type=jnp.float32)

    @pl.when(i == nb - 1)
    def _():
      p = jnp.maximum(
          jnp.dot(acc[...], lw_ref[...], preferred_element_type=jnp.float32)
          + lb_ref[...], 0.0)
      t = jnp.dot(p, fw_ref[...], preferred_element_type=jnp.float32) \
          + fb_ref[...]
      out_ref[...] = jnp.where(t >= 0.0, t, 0.01 * t)

  return pl.pallas_call(
      body,
      grid=(nb,),
      in_specs=[
          pl.BlockSpec((rblk, d), lambda i: (i, 0)),
          pl.BlockSpec((rblk, d), lambda i: (i, 0)),
          pl.BlockSpec((rblk, d), lambda i: (i, 0)),
          pl.BlockSpec((1, 1, rblk), lambda i: (i, 0, 0)),
          pl.BlockSpec((d, d), lambda i: (0, 0)),
          pl.BlockSpec((1, d), lambda i: (0, 0)),
          pl.BlockSpec((d, d), lambda i: (0, 0)),
          pl.BlockSpec((1, d), lambda i: (0, 0)),
          pl.BlockSpec((d, d), lambda i: (0, 0)),
          pl.BlockSpec((1, d), lambda i: (0, 0)),
          pl.BlockSpec((d, d), lambda i: (0, 0)),
          pl.BlockSpec((1, d), lambda i: (0, 0)),
      ],
      out_specs=pl.BlockSpec((g, d), lambda i: (0, 0)),
      out_shape=jax.ShapeDtypeStruct((g, d), jnp.float32),
      scratch_shapes=[pltpu.VMEM((g, d), jnp.float32)],
  )(x, agg0, agg1, batch3, W1f, b1f, W2, b2, lin_W, lin_b, fin_Wp, fin_bp)


def kernel(x, edge_index, batch, W1, b1, bn_gamma, bn_beta, bn_mean, bn_var,
           W2, b2, lin_W, lin_b, fin_W, fin_b):
  n, d = x.shape
  e = edge_index.shape[1]
  g = 64
  rblk = 1000

  src = edge_index[0]
  dst = edge_index[1]
  zeros = jnp.zeros((n, d), jnp.float32)

  agg0, agg1 = _sc_aggregate(x, src, dst, zeros, n, d, e)

  scale = bn_gamma * lax.rsqrt(bn_var + 1e-5)
  W1f = W1 * scale[None, :]
  b1f = (b1 - bn_mean) * scale + bn_beta

  batch3 = batch.reshape(n // rblk, 1, rblk)
  fin_Wp = jnp.zeros((d, d), jnp.float32).at[:, :1].set(fin_W)
  fin_bp = jnp.zeros((1, d), jnp.float32).at[:, :1].set(fin_b[None, :])

  out = _tc_mlp(x, agg0, agg1, batch3, W1f, b1f[None, :], W2, b2[None, :],
                lin_W, lin_b[None, :], fin_Wp, fin_bp, n, d, g, rblk)
  return out[:, :1]
